# baseline (speedup 1.0000x reference)
import functools

import jax
import jax.numpy as jnp
from jax import lax
from jax.experimental import pallas as pl
from jax.experimental.pallas import tpu as pltpu
from jax.experimental.pallas import tpu_sc as plsc


def _gather_rows(node_state, idx, *, nw, g, q):
    n, d = node_state.shape
    r = idx.shape[0]
    per_w = r // nw
    f = g * q
    chunks = per_w // g
    fills = (chunks // q) & ~1
    tail_chunks = chunks - fills * q
    mesh = plsc.VectorSubcoreMesh(core_axis_name="c", subcore_axis_name="s")

    @functools.partial(
        pl.kernel,
        mesh=mesh,
        out_type=jax.ShapeDtypeStruct((r, d), jnp.float32),
        scratch_types=[
            pltpu.VMEM((chunks, g), jnp.int32),
            pltpu.VMEM((f, d), jnp.float32),
            pltpu.VMEM((f, d), jnp.float32),
            pltpu.SemaphoreType.DMA,
            pltpu.SemaphoreType.DMA,
        ],
    )
    def k(node_hbm, idx_hbm, out_hbm, idx_v, buf0, buf1, sem0, sem1):
        nc = 2
        wid = lax.axis_index("s") * nc + lax.axis_index("c")
        pltpu.sync_copy(idx_hbm.at[wid], idx_v)
        base = wid * per_w

        def fire(fill, nq, buf, sem):
            for j in range(nq):
                pltpu.async_copy(node_hbm.at[idx_v.at[fill * q + j]],
                                 buf.at[pl.ds(j * g, g)], sem)

        def drain(fill, nq, buf, sem):
            for j in range(nq):
                pltpu.make_async_copy(node_hbm.at[idx_v.at[fill * q + j]],
                                      buf.at[pl.ds(j * g, g)], sem).wait()

        def write(fill, rows, buf):
            pltpu.sync_copy(buf.at[pl.ds(0, rows)],
                            out_hbm.at[pl.ds(base + fill * f, rows)])

        fire(0, q, buf0, sem0)

        def body(i, carry):
            fire(2 * i + 1, q, buf1, sem1)
            drain(2 * i, q, buf0, sem0)
            write(2 * i, f, buf0)

            @pl.when(i != fills // 2 - 1)
            def _():
                fire(2 * i + 2, q, buf0, sem0)

            drain(2 * i + 1, q, buf1, sem1)
            write(2 * i + 1, f, buf1)
            return carry

        lax.fori_loop(0, fills // 2, body, 0, unroll=False)

        if tail_chunks:
            fire(fills, tail_chunks, buf0, sem0)
            drain(fills, tail_chunks, buf0, sem0)
            write(fills, tail_chunks * g, buf0)

    return k(node_state, idx.reshape(nw, chunks, g))


def kernel(node_state, edge_src, edge_dst):
    e = edge_src.shape[0]
    d = node_state.shape[1]
    nw = 32
    g = 80
    q = 2
    assert (2 * e) % (nw * g) == 0
    idx = jnp.stack([edge_src.astype(jnp.int32),
                     edge_dst.astype(jnp.int32)], axis=1).reshape(-1)
    out = _gather_rows(node_state, idx, nw=nw, g=g, q=q)
    return out.reshape(e, 2 * d)


# speedup vs baseline: 3.0070x; 3.0070x over previous
import functools

import jax
import jax.numpy as jnp
from jax import lax
from jax.experimental import pallas as pl
from jax.experimental.pallas import tpu as pltpu
from jax.experimental.pallas import tpu_sc as plsc


def _gather_incident(node_state, edge_src, edge_dst, *, nw, g):
    n, d = node_state.shape
    e = edge_src.shape[0]
    per_w = e // nw
    fills = per_w // g
    jmax = (fills - 1) // 3
    assert fills == 3 * jmax + 1
    mesh = plsc.VectorSubcoreMesh(core_axis_name="c", subcore_axis_name="s")

    @functools.partial(
        pl.kernel,
        mesh=mesh,
        out_type=jax.ShapeDtypeStruct((e, 2 * d), jnp.float32),
        scratch_types=[
            pltpu.VMEM((fills, g), jnp.int32),
            pltpu.VMEM((fills, g), jnp.int32),
        ] + [pltpu.VMEM((g, d), jnp.float32) for _ in range(6)]
          + [pltpu.SemaphoreType.DMA for _ in range(6)],
    )
    def k(node_hbm, src_hbm, dst_hbm, out_hbm, sidx_v, didx_v,
          sb0, db0, sb1, db1, sb2, db2, g0, g1, g2, w0, w1, w2):
        nc = 2
        wid = lax.axis_index("s") * nc + lax.axis_index("c")
        pltpu.sync_copy(src_hbm.at[wid], sidx_v)
        pltpu.sync_copy(dst_hbm.at[wid], didx_v)
        base = wid * per_w
        slots = [(sb0, db0, g0, w0), (sb1, db1, g1, w1), (sb2, db2, g2, w2)]

        def fire_g(fi, s):
            pltpu.async_copy(node_hbm.at[sidx_v.at[fi]], s[0], s[2])
            pltpu.async_copy(node_hbm.at[didx_v.at[fi]], s[1], s[2])

        def drain_g(fi, s):
            pltpu.make_async_copy(node_hbm.at[sidx_v.at[fi]], s[0], s[2]).wait()
            pltpu.make_async_copy(node_hbm.at[didx_v.at[fi]], s[1], s[2]).wait()

        def outs(fi, s):
            r0 = base + fi * g
            return ((s[0], out_hbm.at[pl.ds(r0, g), pl.ds(0, d)]),
                    (s[1], out_hbm.at[pl.ds(r0, g), pl.ds(d, d)]))

        def write_w(fi, s):
            for src, dst in outs(fi, s):
                pltpu.async_copy(src, dst, s[3])

        def wait_w(fi, s):
            for src, dst in outs(fi, s):
                pltpu.make_async_copy(src, dst, s[3]).wait()

        fire_g(0, slots[0])
        fire_g(1, slots[1])

        def body(j, carry):
            f0 = 3 * j
            drain_g(f0, slots[0])
            write_w(f0, slots[0])

            @pl.when(j > 0)
            def _():
                wait_w(f0 - 1, slots[2])
            fire_g(f0 + 2, slots[2])

            drain_g(f0 + 1, slots[1])
            write_w(f0 + 1, slots[1])
            wait_w(f0, slots[0])
            fire_g(f0 + 3, slots[0])

            drain_g(f0 + 2, slots[2])
            write_w(f0 + 2, slots[2])

            @pl.when(j != jmax - 1)
            def _():
                wait_w(f0 + 1, slots[1])
                fire_g(f0 + 4, slots[1])
            return carry

        lax.fori_loop(0, jmax, body, 0, unroll=False)

        last = 3 * jmax
        drain_g(last, slots[0])
        write_w(last, slots[0])
        wait_w(last - 2, slots[1])
        wait_w(last - 1, slots[2])
        wait_w(last, slots[0])

    src_r = edge_src.astype(jnp.int32).reshape(nw, fills, g)
    dst_r = edge_dst.astype(jnp.int32).reshape(nw, fills, g)
    return k(node_state, src_r, dst_r)


def kernel(node_state, edge_src, edge_dst):
    e = edge_src.shape[0]
    nw = 32
    g = 40
    assert e % (nw * g) == 0 and (e // (nw * g) - 1) % 3 == 0
    return _gather_incident(node_state, edge_src, edge_dst, nw=nw, g=g)


# speedup vs baseline: 3.2139x; 1.0688x over previous
import functools

import jax
import jax.numpy as jnp
from jax import lax
from jax.experimental import pallas as pl
from jax.experimental.pallas import tpu as pltpu
from jax.experimental.pallas import tpu_sc as plsc


def _gather_incident(node_state, edge_src, edge_dst, *, nw, g):
    n, d = node_state.shape
    e = edge_src.shape[0]
    per_w = e // nw
    fills = (per_w // g) & ~1
    tail = per_w - fills * g
    assert tail == 0 or tail % 8 == 0
    mesh = plsc.VectorSubcoreMesh(core_axis_name="c", subcore_axis_name="s")

    @functools.partial(
        pl.kernel,
        mesh=mesh,
        out_type=jax.ShapeDtypeStruct((e, 2 * d), jnp.float32),
        scratch_types=[
            pltpu.VMEM((per_w,), jnp.int32),
            pltpu.VMEM((per_w,), jnp.int32),
            pltpu.VMEM((g, d), jnp.float32),
            pltpu.VMEM((g, d), jnp.float32),
            pltpu.VMEM((g, d), jnp.float32),
            pltpu.VMEM((g, d), jnp.float32),
            pltpu.SemaphoreType.DMA,
            pltpu.SemaphoreType.DMA,
        ],
    )
    def k(node_hbm, src_hbm, dst_hbm, out_hbm, sidx_v, didx_v,
          sbuf0, dbuf0, sbuf1, dbuf1, sem0, sem1):
        nc = 2
        wid = lax.axis_index("s") * nc + lax.axis_index("c")
        pltpu.sync_copy(src_hbm.at[wid], sidx_v)
        pltpu.sync_copy(dst_hbm.at[wid], didx_v)
        base = wid * per_w

        def fire(fill, rows, sbuf, dbuf, sem):
            i0 = fill * g
            pltpu.async_copy(node_hbm.at[sidx_v.at[pl.ds(i0, rows)]],
                             sbuf.at[pl.ds(0, rows)], sem)
            pltpu.async_copy(node_hbm.at[didx_v.at[pl.ds(i0, rows)]],
                             dbuf.at[pl.ds(0, rows)], sem)

        def drain(fill, rows, sbuf, dbuf, sem):
            i0 = fill * g
            pltpu.make_async_copy(node_hbm.at[sidx_v.at[pl.ds(i0, rows)]],
                                  sbuf.at[pl.ds(0, rows)], sem).wait()
            pltpu.make_async_copy(node_hbm.at[didx_v.at[pl.ds(i0, rows)]],
                                  dbuf.at[pl.ds(0, rows)], sem).wait()

        def write(fill, rows, sbuf, dbuf):
            r0 = base + fill * g
            pltpu.sync_copy(sbuf.at[pl.ds(0, rows)],
                            out_hbm.at[pl.ds(r0, rows), pl.ds(0, d)])
            pltpu.sync_copy(dbuf.at[pl.ds(0, rows)],
                            out_hbm.at[pl.ds(r0, rows), pl.ds(d, d)])

        fire(0, g, sbuf0, dbuf0, sem0)

        def body(i, carry):
            fire(2 * i + 1, g, sbuf1, dbuf1, sem1)
            drain(2 * i, g, sbuf0, dbuf0, sem0)
            write(2 * i, g, sbuf0, dbuf0)

            @pl.when(i != fills // 2 - 1)
            def _():
                fire(2 * i + 2, g, sbuf0, dbuf0, sem0)

            drain(2 * i + 1, g, sbuf1, dbuf1, sem1)
            write(2 * i + 1, g, sbuf1, dbuf1)
            return carry

        lax.fori_loop(0, fills // 2, body, 0, unroll=False)

        if tail:
            fire(fills, tail, sbuf0, dbuf0, sem0)
            drain(fills, tail, sbuf0, dbuf0, sem0)
            write(fills, tail, sbuf0, dbuf0)

    src_r = edge_src.astype(jnp.int32).reshape(nw, per_w)
    dst_r = edge_dst.astype(jnp.int32).reshape(nw, per_w)
    return k(node_state, src_r, dst_r)


def kernel(node_state, edge_src, edge_dst):
    e = edge_src.shape[0]
    nw = 32
    g = 128
    assert e % (nw * 8) == 0
    return _gather_incident(node_state, edge_src, edge_dst, nw=nw, g=g)


# speedup vs baseline: 3.2254x; 1.0036x over previous
import functools

import jax
import jax.numpy as jnp
from jax import lax
from jax.experimental import pallas as pl
from jax.experimental.pallas import tpu as pltpu
from jax.experimental.pallas import tpu_sc as plsc


def _gather_incident(node_state, edge_src, edge_dst, *, nw, g):
    n, d = node_state.shape
    e = edge_src.shape[0]
    per_w = e // nw
    fills = (per_w // g) & ~1
    tail = per_w - fills * g
    assert tail == 0 or tail % 8 == 0
    mesh = plsc.VectorSubcoreMesh(core_axis_name="c", subcore_axis_name="s")

    @functools.partial(
        pl.kernel,
        mesh=mesh,
        out_type=jax.ShapeDtypeStruct((e, 2 * d), jnp.float32),
        scratch_types=[
            pltpu.VMEM((per_w,), jnp.int32),
            pltpu.VMEM((per_w,), jnp.int32),
            pltpu.VMEM((g, d), jnp.float32),
            pltpu.VMEM((g, d), jnp.float32),
            pltpu.VMEM((g, d), jnp.float32),
            pltpu.VMEM((g, d), jnp.float32),
            pltpu.SemaphoreType.DMA,
            pltpu.SemaphoreType.DMA,
            pltpu.SemaphoreType.DMA,
        ],
    )
    def k(node_hbm, src_hbm, dst_hbm, out_hbm, sidx_v, didx_v,
          sbuf0, dbuf0, sbuf1, dbuf1, sem0, sem1, wsem):
        nc = 2
        wid = lax.axis_index("s") * nc + lax.axis_index("c")
        pltpu.sync_copy(src_hbm.at[wid], sidx_v)
        pltpu.sync_copy(dst_hbm.at[wid], didx_v)
        base = wid * per_w

        def fire(fill, rows, sbuf, dbuf, sem):
            i0 = fill * g
            pltpu.async_copy(node_hbm.at[sidx_v.at[pl.ds(i0, rows)]],
                             sbuf.at[pl.ds(0, rows)], sem)
            pltpu.async_copy(node_hbm.at[didx_v.at[pl.ds(i0, rows)]],
                             dbuf.at[pl.ds(0, rows)], sem)

        def drain(fill, rows, sbuf, dbuf, sem):
            i0 = fill * g
            pltpu.make_async_copy(node_hbm.at[sidx_v.at[pl.ds(i0, rows)]],
                                  sbuf.at[pl.ds(0, rows)], sem).wait()
            pltpu.make_async_copy(node_hbm.at[didx_v.at[pl.ds(i0, rows)]],
                                  dbuf.at[pl.ds(0, rows)], sem).wait()

        def write(fill, rows, sbuf, dbuf):
            r0 = base + fill * g
            a = pltpu.async_copy(sbuf.at[pl.ds(0, rows)],
                                 out_hbm.at[pl.ds(r0, rows), pl.ds(0, d)],
                                 wsem)
            b = pltpu.async_copy(dbuf.at[pl.ds(0, rows)],
                                 out_hbm.at[pl.ds(r0, rows), pl.ds(d, d)],
                                 wsem)
            a.wait()
            b.wait()

        fire(0, g, sbuf0, dbuf0, sem0)

        def body(i, carry):
            fire(2 * i + 1, g, sbuf1, dbuf1, sem1)
            drain(2 * i, g, sbuf0, dbuf0, sem0)
            write(2 * i, g, sbuf0, dbuf0)

            @pl.when(i != fills // 2 - 1)
            def _():
                fire(2 * i + 2, g, sbuf0, dbuf0, sem0)

            drain(2 * i + 1, g, sbuf1, dbuf1, sem1)
            write(2 * i + 1, g, sbuf1, dbuf1)
            return carry

        lax.fori_loop(0, fills // 2, body, 0, unroll=False)

        if tail:
            fire(fills, tail, sbuf0, dbuf0, sem0)
            drain(fills, tail, sbuf0, dbuf0, sem0)
            write(fills, tail, sbuf0, dbuf0)

    src_r = edge_src.astype(jnp.int32).reshape(nw, per_w)
    dst_r = edge_dst.astype(jnp.int32).reshape(nw, per_w)
    return k(node_state, src_r, dst_r)


def kernel(node_state, edge_src, edge_dst):
    e = edge_src.shape[0]
    nw = 32
    g = 128
    assert e % (nw * 8) == 0
    return _gather_incident(node_state, edge_src, edge_dst, nw=nw, g=g)
